# baseline (speedup 1.0000x reference)
import functools

import jax
import jax.numpy as jnp
from jax import lax
from jax.experimental import pallas as pl
from jax.experimental.pallas import tpu as pltpu
from jax.experimental.pallas import tpu_sc as plsc

N = 10000
E = 160000
D_EDGE = 4
KER_W = 256
W = 32
DEPTH = 4

NC, NS = 2, 16
NW = NC * NS
EPW = E // NW
CH = 125
NCH = EPW // CH
RPT = N // NS

_SC_MESH = plsc.VectorSubcoreMesh(
    core_axis_name="c", subcore_axis_name="s", num_cores=NC, num_subcores=NS)


def _tmlp_body(ea_ref, w1_ref, b1_ref, w2_ref, b2_ref, out_ref):
    t = jnp.dot(ea_ref[...], w1_ref[...], preferred_element_type=jnp.float32)
    t = jnp.maximum(t + b1_ref[...], 0.0)
    t = jnp.dot(t, w2_ref[...], preferred_element_type=jnp.float32)
    out_ref[...] = jnp.maximum(t + b2_ref[...], 0.0)


def _edge_t(edge_attr, kW1, kb1, kW2, kb2, block_e=2000):
    grid = (E // block_e,)
    return pl.pallas_call(
        _tmlp_body,
        grid=grid,
        in_specs=[
            pl.BlockSpec((block_e, D_EDGE), lambda i: (i, 0)),
            pl.BlockSpec((D_EDGE, KER_W), lambda i: (0, 0)),
            pl.BlockSpec((1, KER_W), lambda i: (0, 0)),
            pl.BlockSpec((KER_W, KER_W), lambda i: (0, 0)),
            pl.BlockSpec((1, KER_W), lambda i: (0, 0)),
        ],
        out_specs=pl.BlockSpec((block_e, KER_W), lambda i: (i, 0)),
        out_shape=jax.ShapeDtypeStruct((E, KER_W), jnp.float32),
    )(edge_attr, kW1, kb1, kW2, kb2)


def _h0_body(x_ref, w_ref, b_ref, out_ref):
    out_ref[...] = x_ref[...] * w_ref[...] + b_ref[...]


def _h0(x, fc1_W, fc1_b):
    return pl.pallas_call(
        _h0_body,
        grid=(1,),
        in_specs=[
            pl.BlockSpec((N, 1), lambda i: (0, 0)),
            pl.BlockSpec((1, W), lambda i: (0, 0)),
            pl.BlockSpec((1, W), lambda i: (0, 0)),
        ],
        out_specs=pl.BlockSpec((N, W), lambda i: (0, 0)),
        out_shape=jax.ShapeDtypeStruct((N, W), jnp.float32),
    )(x, fc1_W, fc1_b)


def _msg_body(xj_ref, t_ref, k_hi_ref, k_lo_ref, kb_ref, sel_ref, out_ref):
    t = t_ref[...]
    t_hi = t.astype(jnp.bfloat16)
    t_lo = (t - t_hi.astype(jnp.float32)).astype(jnp.bfloat16)
    k_hi = k_hi_ref[...]
    u = (jnp.dot(t_hi, k_hi, preferred_element_type=jnp.float32)
         + jnp.dot(t_lo, k_hi, preferred_element_type=jnp.float32)
         + jnp.dot(t_hi, k_lo_ref[...], preferred_element_type=jnp.float32)
         + kb_ref[...])
    xq = jnp.tile(xj_ref[...], (1, W))
    p = u * xq
    p_hi = p.astype(jnp.bfloat16)
    p_lo = (p - p_hi.astype(jnp.float32)).astype(jnp.bfloat16)
    sel = sel_ref[...]
    out_ref[...] = (
        jnp.dot(p_hi, sel, preferred_element_type=jnp.float32)
        + jnp.dot(p_lo, sel, preferred_element_type=jnp.float32))


def _msg(xj, t, k_hi, k_lo, kb3p, sel, block_e=2000):
    grid = (E // block_e,)
    return pl.pallas_call(
        _msg_body,
        grid=grid,
        in_specs=[
            pl.BlockSpec((block_e, W), lambda i: (i, 0)),
            pl.BlockSpec((block_e, KER_W), lambda i: (i, 0)),
            pl.BlockSpec((KER_W, W * W), lambda i: (0, 0)),
            pl.BlockSpec((KER_W, W * W), lambda i: (0, 0)),
            pl.BlockSpec((1, W * W), lambda i: (0, 0)),
            pl.BlockSpec((W * W, W), lambda i: (0, 0)),
        ],
        out_specs=pl.BlockSpec((block_e, W), lambda i: (i, 0)),
        out_shape=jax.ShapeDtypeStruct((E, W), jnp.float32),
    )(xj, t, k_hi, k_lo, kb3p, sel)


def _deginv_body(deg_ref, out_ref):
    d = deg_ref[0] + deg_ref[1]
    out_ref[...] = 1.0 / jnp.maximum(d, 1.0)


def _deginv(deg2):
    return pl.pallas_call(
        _deginv_body,
        grid=(1,),
        in_specs=[pl.BlockSpec((NC, N, W), lambda i: (0, 0, 0))],
        out_specs=pl.BlockSpec((N, W), lambda i: (0, 0)),
        out_shape=jax.ShapeDtypeStruct((N, W), jnp.float32),
    )(deg2)


def _update_body(agg_ref, dinv_ref, h_ref, root_ref, b_ref, out_ref, *, relu):
    a = (agg_ref[0] + agg_ref[1]) * dinv_ref[...]
    hn = a + jnp.dot(h_ref[...], root_ref[...],
                     preferred_element_type=jnp.float32) + b_ref[...]
    out_ref[...] = jnp.maximum(hn, 0.0) if relu else hn


def _update(agg2, dinv, h, root, conv_b, relu):
    return pl.pallas_call(
        functools.partial(_update_body, relu=relu),
        grid=(1,),
        in_specs=[
            pl.BlockSpec((NC, N, W), lambda i: (0, 0, 0)),
            pl.BlockSpec((N, W), lambda i: (0, 0)),
            pl.BlockSpec((N, W), lambda i: (0, 0)),
            pl.BlockSpec((W, W), lambda i: (0, 0)),
            pl.BlockSpec((1, W), lambda i: (0, 0)),
        ],
        out_specs=pl.BlockSpec((N, W), lambda i: (0, 0)),
        out_shape=jax.ShapeDtypeStruct((N, W), jnp.float32),
    )(agg2, dinv, h, root, conv_b)


def _final_body(agg_ref, dinv_ref, h_ref, root_ref, b_ref, w2_ref, b2_ref,
                out_ref):
    a = (agg_ref[0] + agg_ref[1]) * dinv_ref[...]
    hn = a + jnp.dot(h_ref[...], root_ref[...],
                     preferred_element_type=jnp.float32) + b_ref[...]
    out_ref[...] = jnp.dot(hn, w2_ref[...],
                           preferred_element_type=jnp.float32) + b2_ref[...]


def _final(agg2, dinv, h, root, conv_b, fc2_W, fc2_b):
    return pl.pallas_call(
        _final_body,
        grid=(1,),
        in_specs=[
            pl.BlockSpec((NC, N, W), lambda i: (0, 0, 0)),
            pl.BlockSpec((N, W), lambda i: (0, 0)),
            pl.BlockSpec((N, W), lambda i: (0, 0)),
            pl.BlockSpec((W, W), lambda i: (0, 0)),
            pl.BlockSpec((1, W), lambda i: (0, 0)),
            pl.BlockSpec((W, 1), lambda i: (0, 0)),
            pl.BlockSpec((1, 1), lambda i: (0, 0)),
        ],
        out_specs=pl.BlockSpec((N, 1), lambda i: (0, 0)),
        out_shape=jax.ShapeDtypeStruct((N, 1), jnp.float32),
    )(agg2, dinv, h, root, conv_b, fc2_W, fc2_b)


GCH = 10
NG = NCH // GCH


def _gather_body(h_hbm, src_hbm, out_hbm, idx2, rows0, rows1, sg0, sg1, sw0,
                 sw1):
    wid = lax.axis_index("s") * NC + lax.axis_index("c")
    pltpu.sync_copy(src_hbm.at[wid], idx2)
    bufs, gsems, wsems = (rows0, rows1), (sg0, sg1), (sw0, sw1)
    writes = [None, None]
    for g in range(NG):
        b = g % 2
        if writes[b] is not None:
            writes[b].wait()
        copies = [
            pltpu.async_copy(h_hbm.at[idx2.at[g * GCH + j]], bufs[b].at[j],
                             gsems[b])
            for j in range(GCH)
        ]
        for cp in copies:
            cp.wait()
        writes[b] = pltpu.async_copy(
            bufs[b], out_hbm.at[wid, pl.ds(g * GCH, GCH)], wsems[b])
    for wr in writes:
        if wr is not None:
            wr.wait()


_gather = pl.kernel(
    _gather_body,
    out_type=jax.ShapeDtypeStruct((NW, NCH, CH, W), jnp.float32),
    mesh=_SC_MESH,
    compiler_params=pltpu.CompilerParams(use_tc_tiling_on_sc=False),
    scratch_types=[
        pltpu.VMEM((NCH, CH), jnp.int32),
        pltpu.VMEM((GCH, CH, W), jnp.float32),
        pltpu.VMEM((GCH, CH, W), jnp.float32),
        pltpu.SemaphoreType.DMA,
        pltpu.SemaphoreType.DMA,
        pltpu.SemaphoreType.DMA,
        pltpu.SemaphoreType.DMA,
    ],
)


def _scatter_body(msg_hbm, dst_hbm, zeros_hbm, out_hbm, idx2, msg0, msg1,
                  agg_sh, sl0, sl1, ss0, ss1):
    cid = lax.axis_index("c")
    sid = lax.axis_index("s")
    wid = sid * NC + cid
    pltpu.sync_copy(zeros_hbm.at[pl.ds(sid * RPT, RPT)],
                    agg_sh.at[pl.ds(sid * RPT, RPT)])
    pltpu.sync_copy(dst_hbm.at[wid], idx2)
    plsc.subcore_barrier()

    bufs, lsems, ssems = (msg0, msg1), (sl0, sl1), (ss0, ss1)
    loads = [None, None]
    adds = [[], []]
    loads[0] = pltpu.async_copy(msg_hbm.at[wid, pl.ds(0, GCH)], bufs[0],
                                lsems[0])
    for g in range(NG):
        b = g % 2
        nb = (g + 1) % 2
        if g + 1 < NG:
            for cp in adds[nb]:
                cp.wait()
            loads[nb] = pltpu.async_copy(
                msg_hbm.at[wid, pl.ds((g + 1) * GCH, GCH)], bufs[nb],
                lsems[nb])
        loads[b].wait()
        adds[b] = [
            pltpu.async_copy(bufs[b].at[j], agg_sh.at[idx2.at[g * GCH + j]],
                             ssems[b], add=True)
            for j in range(GCH)
        ]
    for cps in adds:
        for cp in cps:
            cp.wait()
    plsc.subcore_barrier()
    pltpu.sync_copy(agg_sh.at[pl.ds(sid * RPT, RPT)],
                    out_hbm.at[cid, pl.ds(sid * RPT, RPT)])


_scatter = pl.kernel(
    _scatter_body,
    out_type=jax.ShapeDtypeStruct((NC, N, W), jnp.float32),
    mesh=_SC_MESH,
    compiler_params=pltpu.CompilerParams(use_tc_tiling_on_sc=False),
    scratch_types=[
        pltpu.VMEM((NCH, CH), jnp.int32),
        pltpu.VMEM((GCH, CH, W), jnp.float32),
        pltpu.VMEM((GCH, CH, W), jnp.float32),
        pltpu.VMEM_SHARED((N, W), jnp.float32),
        pltpu.SemaphoreType.DMA,
        pltpu.SemaphoreType.DMA,
        pltpu.SemaphoreType.DMA,
        pltpu.SemaphoreType.DMA,
    ],
)


def _degree_body(dst_hbm, ones_hbm, zeros_hbm, out_hbm, idx_v, ones_v, agg_sh):
    cid = lax.axis_index("c")
    sid = lax.axis_index("s")
    wid = sid * NC + cid
    pltpu.sync_copy(zeros_hbm.at[pl.ds(sid * RPT, RPT)],
                    agg_sh.at[pl.ds(sid * RPT, RPT)])
    pltpu.sync_copy(ones_hbm, ones_v)
    plsc.subcore_barrier()

    def chunk(c, carry):
        pltpu.sync_copy(dst_hbm.at[wid, c], idx_v)
        pltpu.sync_copy(ones_v, agg_sh.at[idx_v], add=True)
        return carry

    lax.fori_loop(0, NCH, chunk, 0)
    plsc.subcore_barrier()
    pltpu.sync_copy(agg_sh.at[pl.ds(sid * RPT, RPT)],
                    out_hbm.at[cid, pl.ds(sid * RPT, RPT)])


_degree = pl.kernel(
    _degree_body,
    out_type=jax.ShapeDtypeStruct((NC, N, W), jnp.float32),
    mesh=_SC_MESH,
    compiler_params=pltpu.CompilerParams(use_tc_tiling_on_sc=False),
    scratch_types=[
        pltpu.VMEM((CH,), jnp.int32),
        pltpu.VMEM((CH, W), jnp.float32),
        pltpu.VMEM_SHARED((N, W), jnp.float32),
    ],
)


def kernel(x, edge_index, edge_attr, fc1_W, fc1_b, kW1, kb1, kW2, kb2, kW3,
           kb3, root, conv_b, fc2_W, fc2_b):
    src3 = edge_index[0].reshape(NW, NCH, CH)
    dst3 = edge_index[1].reshape(NW, NCH, CH)
    kW3p = kW3.reshape(KER_W, W, W).transpose(0, 2, 1).reshape(KER_W, W * W)
    kb3p = kb3.reshape(W, W).T.reshape(1, W * W)
    zeros = jnp.zeros((N, W), jnp.float32)
    ones_ch = jnp.ones((CH, W), jnp.float32)
    col = jnp.arange(W * W, dtype=jnp.int32)
    sel = (col[:, None] // W == jnp.arange(W, dtype=jnp.int32)[None, :]
           ).astype(jnp.bfloat16)

    k_hi = kW3p.astype(jnp.bfloat16)
    k_lo = (kW3p - k_hi.astype(jnp.float32)).astype(jnp.bfloat16)
    t = _edge_t(edge_attr, kW1, kb1.reshape(1, KER_W), kW2,
                kb2.reshape(1, KER_W))
    h = _h0(x, fc1_W, fc1_b.reshape(1, W))
    deg2 = _degree(dst3, ones_ch, zeros)
    dinv = _deginv(deg2)
    conv_br = conv_b.reshape(1, W)

    out = None
    for d in range(DEPTH):
        xj = _gather(h, src3).reshape(E, W)
        msg = _msg(xj, t, k_hi, k_lo, kb3p, sel).reshape(NW, NCH, CH, W)
        agg2 = _scatter(msg, dst3, zeros)
        if d < DEPTH - 1:
            h = _update(agg2, dinv, h, root, conv_br, relu=True)
        else:
            out = _final(agg2, dinv, h, root, conv_br, fc2_W,
                         fc2_b.reshape(1, 1))
    return out


# speedup vs baseline: 1.6294x; 1.6294x over previous
import functools

import jax
import jax.numpy as jnp
from jax import lax
from jax.experimental import pallas as pl
from jax.experimental.pallas import tpu as pltpu
from jax.experimental.pallas import tpu_sc as plsc

N = 10000
E = 160000
D_EDGE = 4
KER_W = 256
W = 32
DEPTH = 4

NC, NS = 2, 16
NW = NC * NS
EPW = E // NW
CH = 125
NCH = EPW // CH
RPT = N // NS

_SC_MESH = plsc.VectorSubcoreMesh(
    core_axis_name="c", subcore_axis_name="s", num_cores=NC, num_subcores=NS)


def _wmlp_body(ea_ref, w1_ref, b1_ref, w2_ref, b2_ref, w3_ref, b3_ref, out_ref):
    t = jnp.dot(ea_ref[...], w1_ref[...], preferred_element_type=jnp.float32)
    t = jnp.maximum(t + b1_ref[...], 0.0)
    t = jnp.dot(t, w2_ref[...], preferred_element_type=jnp.float32)
    t = jnp.maximum(t + b2_ref[...], 0.0)
    out_ref[...] = (
        jnp.dot(t, w3_ref[...], preferred_element_type=jnp.float32) + b3_ref[...])


def _edge_mlp(edge_attr, kW1, kb1, kW2, kb2, kW3p, kb3p, block_e=1000):
    grid = (E // block_e,)
    return pl.pallas_call(
        _wmlp_body,
        grid=grid,
        in_specs=[
            pl.BlockSpec((block_e, D_EDGE), lambda i: (i, 0)),
            pl.BlockSpec((D_EDGE, KER_W), lambda i: (0, 0)),
            pl.BlockSpec((1, KER_W), lambda i: (0, 0)),
            pl.BlockSpec((KER_W, KER_W), lambda i: (0, 0)),
            pl.BlockSpec((1, KER_W), lambda i: (0, 0)),
            pl.BlockSpec((KER_W, W * W), lambda i: (0, 0)),
            pl.BlockSpec((1, W * W), lambda i: (0, 0)),
        ],
        out_specs=pl.BlockSpec((block_e, W * W), lambda i: (i, 0)),
        out_shape=jax.ShapeDtypeStruct((E, W * W), jnp.float32),
    )(edge_attr, kW1, kb1, kW2, kb2, kW3p, kb3p)


def _h0_body(x_ref, w_ref, b_ref, out_ref):
    out_ref[...] = x_ref[...] * w_ref[...] + b_ref[...]


def _h0(x, fc1_W, fc1_b):
    return pl.pallas_call(
        _h0_body,
        grid=(1,),
        in_specs=[
            pl.BlockSpec((N, 1), lambda i: (0, 0)),
            pl.BlockSpec((1, W), lambda i: (0, 0)),
            pl.BlockSpec((1, W), lambda i: (0, 0)),
        ],
        out_specs=pl.BlockSpec((N, W), lambda i: (0, 0)),
        out_shape=jax.ShapeDtypeStruct((N, W), jnp.float32),
    )(x, fc1_W, fc1_b)


def _msg_body(xj_ref, w_ref, sel_ref, out_ref):
    xq = jnp.tile(xj_ref[...], (1, W))
    p = w_ref[...] * xq
    p_hi = p.astype(jnp.bfloat16)
    p_lo = (p - p_hi.astype(jnp.float32)).astype(jnp.bfloat16)
    sel = sel_ref[...]
    out_ref[...] = (
        jnp.dot(p_hi, sel, preferred_element_type=jnp.float32)
        + jnp.dot(p_lo, sel, preferred_element_type=jnp.float32))


def _msg(xj, w, sel, half, block_e=2000):
    nblk = (E // 2) // block_e
    return pl.pallas_call(
        _msg_body,
        grid=(nblk,),
        in_specs=[
            pl.BlockSpec((block_e, W), lambda i: (i, 0)),
            pl.BlockSpec((block_e, W * W),
                         lambda i, h=half, n=nblk: (i + h * n, 0)),
            pl.BlockSpec((W * W, W), lambda i: (0, 0)),
        ],
        out_specs=pl.BlockSpec((block_e, W), lambda i: (i, 0)),
        out_shape=jax.ShapeDtypeStruct((E // 2, W), jnp.float32),
    )(xj, w, sel)


def _deginv_body(deg_ref, out_ref):
    d = deg_ref[0] + deg_ref[1]
    out_ref[...] = 1.0 / jnp.maximum(d, 1.0)


def _deginv(deg2):
    return pl.pallas_call(
        _deginv_body,
        grid=(1,),
        in_specs=[pl.BlockSpec((NC, N, W), lambda i: (0, 0, 0))],
        out_specs=pl.BlockSpec((N, W), lambda i: (0, 0)),
        out_shape=jax.ShapeDtypeStruct((N, W), jnp.float32),
    )(deg2)


def _update_body(aggA_ref, aggB_ref, dinv_ref, h_ref, root_ref, b_ref,
                 out_ref, *, relu):
    a = (aggA_ref[0] + aggA_ref[1] + aggB_ref[0] + aggB_ref[1]) * dinv_ref[...]
    hn = a + jnp.dot(h_ref[...], root_ref[...],
                     preferred_element_type=jnp.float32) + b_ref[...]
    out_ref[...] = jnp.maximum(hn, 0.0) if relu else hn


def _update(aggA, aggB, dinv, h, root, conv_b, relu):
    return pl.pallas_call(
        functools.partial(_update_body, relu=relu),
        grid=(1,),
        in_specs=[
            pl.BlockSpec((NC, N, W), lambda i: (0, 0, 0)),
            pl.BlockSpec((NC, N, W), lambda i: (0, 0, 0)),
            pl.BlockSpec((N, W), lambda i: (0, 0)),
            pl.BlockSpec((N, W), lambda i: (0, 0)),
            pl.BlockSpec((W, W), lambda i: (0, 0)),
            pl.BlockSpec((1, W), lambda i: (0, 0)),
        ],
        out_specs=pl.BlockSpec((N, W), lambda i: (0, 0)),
        out_shape=jax.ShapeDtypeStruct((N, W), jnp.float32),
    )(aggA, aggB, dinv, h, root, conv_b)


def _final_body(aggA_ref, aggB_ref, dinv_ref, h_ref, root_ref, b_ref, w2_ref,
                b2_ref, out_ref):
    a = (aggA_ref[0] + aggA_ref[1] + aggB_ref[0] + aggB_ref[1]) * dinv_ref[...]
    hn = a + jnp.dot(h_ref[...], root_ref[...],
                     preferred_element_type=jnp.float32) + b_ref[...]
    out_ref[...] = jnp.dot(hn, w2_ref[...],
                           preferred_element_type=jnp.float32) + b2_ref[...]


def _final(aggA, aggB, dinv, h, root, conv_b, fc2_W, fc2_b):
    return pl.pallas_call(
        _final_body,
        grid=(1,),
        in_specs=[
            pl.BlockSpec((NC, N, W), lambda i: (0, 0, 0)),
            pl.BlockSpec((NC, N, W), lambda i: (0, 0, 0)),
            pl.BlockSpec((N, W), lambda i: (0, 0)),
            pl.BlockSpec((N, W), lambda i: (0, 0)),
            pl.BlockSpec((W, W), lambda i: (0, 0)),
            pl.BlockSpec((1, W), lambda i: (0, 0)),
            pl.BlockSpec((W, 1), lambda i: (0, 0)),
            pl.BlockSpec((1, 1), lambda i: (0, 0)),
        ],
        out_specs=pl.BlockSpec((N, 1), lambda i: (0, 0)),
        out_shape=jax.ShapeDtypeStruct((N, 1), jnp.float32),
    )(aggA, aggB, dinv, h, root, conv_b, fc2_W, fc2_b)


GCH = 10


def _make_gather(nch):
    ng = nch // GCH

    def body(h_hbm, src_hbm, out_hbm, idx2, rows0, rows1, sg0, sg1, sw0, sw1):
        wid = lax.axis_index("s") * NC + lax.axis_index("c")
        pltpu.sync_copy(src_hbm.at[wid], idx2)
        bufs, gsems, wsems = (rows0, rows1), (sg0, sg1), (sw0, sw1)
        writes = [None, None]
        for g in range(ng):
            b = g % 2
            if writes[b] is not None:
                writes[b].wait()
            copies = [
                pltpu.async_copy(h_hbm.at[idx2.at[g * GCH + j]], bufs[b].at[j],
                                 gsems[b])
                for j in range(GCH)
            ]
            for cp in copies:
                cp.wait()
            writes[b] = pltpu.async_copy(
                bufs[b], out_hbm.at[wid, pl.ds(g * GCH, GCH)], wsems[b])
        for wr in writes:
            if wr is not None:
                wr.wait()

    return pl.kernel(
        body,
        out_type=jax.ShapeDtypeStruct((NW, nch, CH, W), jnp.float32),
        mesh=_SC_MESH,
        compiler_params=pltpu.CompilerParams(use_tc_tiling_on_sc=False),
        scratch_types=[
            pltpu.VMEM((nch, CH), jnp.int32),
            pltpu.VMEM((GCH, CH, W), jnp.float32),
            pltpu.VMEM((GCH, CH, W), jnp.float32),
            pltpu.SemaphoreType.DMA,
            pltpu.SemaphoreType.DMA,
            pltpu.SemaphoreType.DMA,
            pltpu.SemaphoreType.DMA,
        ],
    )


_gather_half = _make_gather(NCH // 2)


def _make_scatter(nch):
    ng = nch // GCH

    def body(msg_hbm, dst_hbm, zeros_hbm, out_hbm, idx2, msg0, msg1, agg_sh,
             sl0, sl1, ss0, ss1):
        cid = lax.axis_index("c")
        sid = lax.axis_index("s")
        wid = sid * NC + cid
        pltpu.sync_copy(zeros_hbm.at[pl.ds(sid * RPT, RPT)],
                        agg_sh.at[pl.ds(sid * RPT, RPT)])
        pltpu.sync_copy(dst_hbm.at[wid], idx2)
        plsc.subcore_barrier()

        bufs, lsems, ssems = (msg0, msg1), (sl0, sl1), (ss0, ss1)
        loads = [None, None]
        adds = [[], []]
        loads[0] = pltpu.async_copy(msg_hbm.at[wid, pl.ds(0, GCH)], bufs[0],
                                    lsems[0])
        for g in range(ng):
            b = g % 2
            nb = (g + 1) % 2
            if g + 1 < ng:
                for cp in adds[nb]:
                    cp.wait()
                loads[nb] = pltpu.async_copy(
                    msg_hbm.at[wid, pl.ds((g + 1) * GCH, GCH)], bufs[nb],
                    lsems[nb])
            loads[b].wait()
            adds[b] = [
                pltpu.async_copy(bufs[b].at[j],
                                 agg_sh.at[idx2.at[g * GCH + j]],
                                 ssems[b], add=True)
                for j in range(GCH)
            ]
        for cps in adds:
            for cp in cps:
                cp.wait()
        plsc.subcore_barrier()
        pltpu.sync_copy(agg_sh.at[pl.ds(sid * RPT, RPT)],
                        out_hbm.at[cid, pl.ds(sid * RPT, RPT)])

    return pl.kernel(
        body,
        out_type=jax.ShapeDtypeStruct((NC, N, W), jnp.float32),
        mesh=_SC_MESH,
        compiler_params=pltpu.CompilerParams(use_tc_tiling_on_sc=False),
        scratch_types=[
            pltpu.VMEM((nch, CH), jnp.int32),
            pltpu.VMEM((GCH, CH, W), jnp.float32),
            pltpu.VMEM((GCH, CH, W), jnp.float32),
            pltpu.VMEM_SHARED((N, W), jnp.float32),
            pltpu.SemaphoreType.DMA,
            pltpu.SemaphoreType.DMA,
            pltpu.SemaphoreType.DMA,
            pltpu.SemaphoreType.DMA,
        ],
    )


_scatter_half = _make_scatter(NCH // 2)


def _degree_body(dst_hbm, ones_hbm, zeros_hbm, out_hbm, idx_v, ones_v, agg_sh):
    cid = lax.axis_index("c")
    sid = lax.axis_index("s")
    wid = sid * NC + cid
    pltpu.sync_copy(zeros_hbm.at[pl.ds(sid * RPT, RPT)],
                    agg_sh.at[pl.ds(sid * RPT, RPT)])
    pltpu.sync_copy(ones_hbm, ones_v)
    plsc.subcore_barrier()

    def chunk(c, carry):
        pltpu.sync_copy(dst_hbm.at[wid, c], idx_v)
        pltpu.sync_copy(ones_v, agg_sh.at[idx_v], add=True)
        return carry

    lax.fori_loop(0, NCH, chunk, 0)
    plsc.subcore_barrier()
    pltpu.sync_copy(agg_sh.at[pl.ds(sid * RPT, RPT)],
                    out_hbm.at[cid, pl.ds(sid * RPT, RPT)])


_degree = pl.kernel(
    _degree_body,
    out_type=jax.ShapeDtypeStruct((NC, N, W), jnp.float32),
    mesh=_SC_MESH,
    compiler_params=pltpu.CompilerParams(use_tc_tiling_on_sc=False),
    scratch_types=[
        pltpu.VMEM((CH,), jnp.int32),
        pltpu.VMEM((CH, W), jnp.float32),
        pltpu.VMEM_SHARED((N, W), jnp.float32),
    ],
)


def kernel(x, edge_index, edge_attr, fc1_W, fc1_b, kW1, kb1, kW2, kb2, kW3,
           kb3, root, conv_b, fc2_W, fc2_b):
    NCH2 = NCH // 2
    EH = E // 2
    src_h = edge_index[0].reshape(2, NW, NCH2, CH)
    dst_h = edge_index[1].reshape(2, NW, NCH2, CH)
    dst3 = edge_index[1].reshape(NW, NCH, CH)
    kW3p = kW3.reshape(KER_W, W, W).transpose(0, 2, 1).reshape(KER_W, W * W)
    kb3p = kb3.reshape(W, W).T.reshape(1, W * W)
    zeros = jnp.zeros((N, W), jnp.float32)
    ones_ch = jnp.ones((CH, W), jnp.float32)
    col = jnp.arange(W * W, dtype=jnp.int32)
    sel = (col[:, None] // W == jnp.arange(W, dtype=jnp.int32)[None, :]
           ).astype(jnp.bfloat16)

    w = _edge_mlp(edge_attr, kW1, kb1.reshape(1, KER_W), kW2,
                  kb2.reshape(1, KER_W), kW3p, kb3p)
    h = _h0(x, fc1_W, fc1_b.reshape(1, W))
    deg2 = _degree(dst3, ones_ch, zeros)
    dinv = _deginv(deg2)
    conv_br = conv_b.reshape(1, W)

    out = None
    for d in range(DEPTH):
        xjA = _gather_half(h, src_h[0]).reshape(EH, W)
        xjB = _gather_half(h, src_h[1]).reshape(EH, W)
        msgA = _msg(xjA, w, sel, 0).reshape(NW, NCH2, CH, W)
        aggA = _scatter_half(msgA, dst_h[0], zeros)
        msgB = _msg(xjB, w, sel, 1).reshape(NW, NCH2, CH, W)
        aggB = _scatter_half(msgB, dst_h[1], zeros)
        if d < DEPTH - 1:
            h = _update(aggA, aggB, dinv, h, root, conv_br, relu=True)
        else:
            out = _final(aggA, aggB, dinv, h, root, conv_br, fc2_W,
                         fc2_b.reshape(1, 1))
    return out


# speedup vs baseline: 1.8737x; 1.1500x over previous
import functools

import jax
import jax.numpy as jnp
from jax import lax
from jax.experimental import pallas as pl
from jax.experimental.pallas import tpu as pltpu
from jax.experimental.pallas import tpu_sc as plsc

N = 10000
E = 160000
D_EDGE = 4
KER_W = 256
W = 32
DEPTH = 4

NC, NS = 2, 16
NW = NC * NS
EPW = E // NW
CH = 125
NCH = EPW // CH
RPT = N // NS

_SC_MESH = plsc.VectorSubcoreMesh(
    core_axis_name="c", subcore_axis_name="s", num_cores=NC, num_subcores=NS)


def _wmlp_body(ea_ref, w1_ref, b1_ref, w2_ref, b2_ref, w3_ref, b3_ref, out_ref):
    t = jnp.dot(ea_ref[...], w1_ref[...], preferred_element_type=jnp.float32)
    t = jnp.maximum(t + b1_ref[...], 0.0)
    t = jnp.dot(t, w2_ref[...], preferred_element_type=jnp.float32)
    t = jnp.maximum(t + b2_ref[...], 0.0)
    out_ref[...] = (
        jnp.dot(t, w3_ref[...], preferred_element_type=jnp.float32)
        + b3_ref[...]).astype(jnp.bfloat16)


def _edge_mlp(edge_attr, kW1, kb1, kW2, kb2, kW3p, kb3p, block_e=1000):
    grid = (E // block_e,)
    return pl.pallas_call(
        _wmlp_body,
        grid=grid,
        in_specs=[
            pl.BlockSpec((block_e, D_EDGE), lambda i: (i, 0)),
            pl.BlockSpec((D_EDGE, KER_W), lambda i: (0, 0)),
            pl.BlockSpec((1, KER_W), lambda i: (0, 0)),
            pl.BlockSpec((KER_W, KER_W), lambda i: (0, 0)),
            pl.BlockSpec((1, KER_W), lambda i: (0, 0)),
            pl.BlockSpec((KER_W, W * W), lambda i: (0, 0)),
            pl.BlockSpec((1, W * W), lambda i: (0, 0)),
        ],
        out_specs=pl.BlockSpec((block_e, W * W), lambda i: (i, 0)),
        out_shape=jax.ShapeDtypeStruct((E, W * W), jnp.bfloat16),
    )(edge_attr, kW1, kb1, kW2, kb2, kW3p, kb3p)


def _h0_body(x_ref, w_ref, b_ref, out_ref):
    out_ref[...] = x_ref[...] * w_ref[...] + b_ref[...]


def _h0(x, fc1_W, fc1_b):
    return pl.pallas_call(
        _h0_body,
        grid=(1,),
        in_specs=[
            pl.BlockSpec((N, 1), lambda i: (0, 0)),
            pl.BlockSpec((1, W), lambda i: (0, 0)),
            pl.BlockSpec((1, W), lambda i: (0, 0)),
        ],
        out_specs=pl.BlockSpec((N, W), lambda i: (0, 0)),
        out_shape=jax.ShapeDtypeStruct((N, W), jnp.float32),
    )(x, fc1_W, fc1_b)


def _msg_body(xj_ref, w_ref, sel_ref, out_ref):
    xq = jnp.tile(xj_ref[...], (1, W))
    p = w_ref[...].astype(jnp.float32) * xq
    p_hi = p.astype(jnp.bfloat16)
    p_lo = (p - p_hi.astype(jnp.float32)).astype(jnp.bfloat16)
    sel = sel_ref[...]
    out_ref[...] = (
        jnp.dot(p_hi, sel, preferred_element_type=jnp.float32)
        + jnp.dot(p_lo, sel, preferred_element_type=jnp.float32))


def _msg(xj, w, sel, block_e=2000):
    grid = (E // block_e,)
    return pl.pallas_call(
        _msg_body,
        grid=grid,
        in_specs=[
            pl.BlockSpec((block_e, W), lambda i: (i, 0)),
            pl.BlockSpec((block_e, W * W), lambda i: (i, 0)),
            pl.BlockSpec((W * W, W), lambda i: (0, 0)),
        ],
        out_specs=pl.BlockSpec((block_e, W), lambda i: (i, 0)),
        out_shape=jax.ShapeDtypeStruct((E, W), jnp.float32),
    )(xj, w, sel)


def _deginv_body(deg_ref, out_ref):
    d = deg_ref[0] + deg_ref[1]
    out_ref[...] = 1.0 / jnp.maximum(d, 1.0)


def _deginv(deg2):
    return pl.pallas_call(
        _deginv_body,
        grid=(1,),
        in_specs=[pl.BlockSpec((NC, N, W), lambda i: (0, 0, 0))],
        out_specs=pl.BlockSpec((N, W), lambda i: (0, 0)),
        out_shape=jax.ShapeDtypeStruct((N, W), jnp.float32),
    )(deg2)


def _update_body(agg_ref, dinv_ref, h_ref, root_ref, b_ref, out_ref, *, relu):
    a = (agg_ref[0] + agg_ref[1]) * dinv_ref[...]
    hn = a + jnp.dot(h_ref[...], root_ref[...],
                     preferred_element_type=jnp.float32) + b_ref[...]
    out_ref[...] = jnp.maximum(hn, 0.0) if relu else hn


def _update(agg2, dinv, h, root, conv_b, relu):
    return pl.pallas_call(
        functools.partial(_update_body, relu=relu),
        grid=(1,),
        in_specs=[
            pl.BlockSpec((NC, N, W), lambda i: (0, 0, 0)),
            pl.BlockSpec((N, W), lambda i: (0, 0)),
            pl.BlockSpec((N, W), lambda i: (0, 0)),
            pl.BlockSpec((W, W), lambda i: (0, 0)),
            pl.BlockSpec((1, W), lambda i: (0, 0)),
        ],
        out_specs=pl.BlockSpec((N, W), lambda i: (0, 0)),
        out_shape=jax.ShapeDtypeStruct((N, W), jnp.float32),
    )(agg2, dinv, h, root, conv_b)


def _final_body(agg_ref, dinv_ref, h_ref, root_ref, b_ref, w2_ref, b2_ref,
                out_ref):
    a = (agg_ref[0] + agg_ref[1]) * dinv_ref[...]
    hn = a + jnp.dot(h_ref[...], root_ref[...],
                     preferred_element_type=jnp.float32) + b_ref[...]
    out_ref[...] = jnp.dot(hn, w2_ref[...],
                           preferred_element_type=jnp.float32) + b2_ref[...]


def _final(agg2, dinv, h, root, conv_b, fc2_W, fc2_b):
    return pl.pallas_call(
        _final_body,
        grid=(1,),
        in_specs=[
            pl.BlockSpec((NC, N, W), lambda i: (0, 0, 0)),
            pl.BlockSpec((N, W), lambda i: (0, 0)),
            pl.BlockSpec((N, W), lambda i: (0, 0)),
            pl.BlockSpec((W, W), lambda i: (0, 0)),
            pl.BlockSpec((1, W), lambda i: (0, 0)),
            pl.BlockSpec((W, 1), lambda i: (0, 0)),
            pl.BlockSpec((1, 1), lambda i: (0, 0)),
        ],
        out_specs=pl.BlockSpec((N, 1), lambda i: (0, 0)),
        out_shape=jax.ShapeDtypeStruct((N, 1), jnp.float32),
    )(agg2, dinv, h, root, conv_b, fc2_W, fc2_b)


GCH = 10
NG = NCH // GCH


def _gather_body(h_hbm, src_hbm, out_hbm, idx2, rows0, rows1, sg0, sg1, sw0,
                 sw1):
    wid = lax.axis_index("s") * NC + lax.axis_index("c")
    pltpu.sync_copy(src_hbm.at[wid], idx2)
    bufs, gsems, wsems = (rows0, rows1), (sg0, sg1), (sw0, sw1)
    writes = [None, None]
    for g in range(NG):
        b = g % 2
        if writes[b] is not None:
            writes[b].wait()
        copies = [
            pltpu.async_copy(h_hbm.at[idx2.at[g * GCH + j]], bufs[b].at[j],
                             gsems[b])
            for j in range(GCH)
        ]
        for cp in copies:
            cp.wait()
        writes[b] = pltpu.async_copy(
            bufs[b], out_hbm.at[wid, pl.ds(g * GCH, GCH)], wsems[b])
    for wr in writes:
        if wr is not None:
            wr.wait()


_gather = pl.kernel(
    _gather_body,
    out_type=jax.ShapeDtypeStruct((NW, NCH, CH, W), jnp.float32),
    mesh=_SC_MESH,
    compiler_params=pltpu.CompilerParams(use_tc_tiling_on_sc=False),
    scratch_types=[
        pltpu.VMEM((NCH, CH), jnp.int32),
        pltpu.VMEM((GCH, CH, W), jnp.float32),
        pltpu.VMEM((GCH, CH, W), jnp.float32),
        pltpu.SemaphoreType.DMA,
        pltpu.SemaphoreType.DMA,
        pltpu.SemaphoreType.DMA,
        pltpu.SemaphoreType.DMA,
    ],
)


def _scatter_body(msg_hbm, dst_hbm, zeros_hbm, out_hbm, idx2, msg0, msg1,
                  agg_sh, sl0, sl1, ss0, ss1):
    cid = lax.axis_index("c")
    sid = lax.axis_index("s")
    wid = sid * NC + cid
    pltpu.sync_copy(zeros_hbm.at[pl.ds(sid * RPT, RPT)],
                    agg_sh.at[pl.ds(sid * RPT, RPT)])
    pltpu.sync_copy(dst_hbm.at[wid], idx2)
    plsc.subcore_barrier()

    bufs, lsems, ssems = (msg0, msg1), (sl0, sl1), (ss0, ss1)
    loads = [None, None]
    adds = [[], []]
    loads[0] = pltpu.async_copy(msg_hbm.at[wid, pl.ds(0, GCH)], bufs[0],
                                lsems[0])
    for g in range(NG):
        b = g % 2
        nb = (g + 1) % 2
        if g + 1 < NG:
            for cp in adds[nb]:
                cp.wait()
            loads[nb] = pltpu.async_copy(
                msg_hbm.at[wid, pl.ds((g + 1) * GCH, GCH)], bufs[nb],
                lsems[nb])
        loads[b].wait()
        adds[b] = [
            pltpu.async_copy(bufs[b].at[j], agg_sh.at[idx2.at[g * GCH + j]],
                             ssems[b], add=True)
            for j in range(GCH)
        ]
    for cps in adds:
        for cp in cps:
            cp.wait()
    plsc.subcore_barrier()
    pltpu.sync_copy(agg_sh.at[pl.ds(sid * RPT, RPT)],
                    out_hbm.at[cid, pl.ds(sid * RPT, RPT)])


_scatter = pl.kernel(
    _scatter_body,
    out_type=jax.ShapeDtypeStruct((NC, N, W), jnp.float32),
    mesh=_SC_MESH,
    compiler_params=pltpu.CompilerParams(use_tc_tiling_on_sc=False),
    scratch_types=[
        pltpu.VMEM((NCH, CH), jnp.int32),
        pltpu.VMEM((GCH, CH, W), jnp.float32),
        pltpu.VMEM((GCH, CH, W), jnp.float32),
        pltpu.VMEM_SHARED((N, W), jnp.float32),
        pltpu.SemaphoreType.DMA,
        pltpu.SemaphoreType.DMA,
        pltpu.SemaphoreType.DMA,
        pltpu.SemaphoreType.DMA,
    ],
)


def _degree_body(dst_hbm, ones_hbm, zeros_hbm, out_hbm, idx_v, ones_v, agg_sh):
    cid = lax.axis_index("c")
    sid = lax.axis_index("s")
    wid = sid * NC + cid
    pltpu.sync_copy(zeros_hbm.at[pl.ds(sid * RPT, RPT)],
                    agg_sh.at[pl.ds(sid * RPT, RPT)])
    pltpu.sync_copy(ones_hbm, ones_v)
    plsc.subcore_barrier()

    def chunk(c, carry):
        pltpu.sync_copy(dst_hbm.at[wid, c], idx_v)
        pltpu.sync_copy(ones_v, agg_sh.at[idx_v], add=True)
        return carry

    lax.fori_loop(0, NCH, chunk, 0)
    plsc.subcore_barrier()
    pltpu.sync_copy(agg_sh.at[pl.ds(sid * RPT, RPT)],
                    out_hbm.at[cid, pl.ds(sid * RPT, RPT)])


_degree = pl.kernel(
    _degree_body,
    out_type=jax.ShapeDtypeStruct((NC, N, W), jnp.float32),
    mesh=_SC_MESH,
    compiler_params=pltpu.CompilerParams(use_tc_tiling_on_sc=False),
    scratch_types=[
        pltpu.VMEM((CH,), jnp.int32),
        pltpu.VMEM((CH, W), jnp.float32),
        pltpu.VMEM_SHARED((N, W), jnp.float32),
    ],
)


def kernel(x, edge_index, edge_attr, fc1_W, fc1_b, kW1, kb1, kW2, kb2, kW3,
           kb3, root, conv_b, fc2_W, fc2_b):
    src3 = edge_index[0].reshape(NW, NCH, CH)
    dst3 = edge_index[1].reshape(NW, NCH, CH)
    kW3p = kW3.reshape(KER_W, W, W).transpose(0, 2, 1).reshape(KER_W, W * W)
    kb3p = kb3.reshape(W, W).T.reshape(1, W * W)
    zeros = jnp.zeros((N, W), jnp.float32)
    ones_ch = jnp.ones((CH, W), jnp.float32)
    col = jnp.arange(W * W, dtype=jnp.int32)
    sel = (col[:, None] // W == jnp.arange(W, dtype=jnp.int32)[None, :]
           ).astype(jnp.bfloat16)

    w = _edge_mlp(edge_attr, kW1, kb1.reshape(1, KER_W), kW2,
                  kb2.reshape(1, KER_W), kW3p, kb3p)
    h = _h0(x, fc1_W, fc1_b.reshape(1, W))
    deg2 = _degree(dst3, ones_ch, zeros)
    dinv = _deginv(deg2)
    conv_br = conv_b.reshape(1, W)

    out = None
    for d in range(DEPTH):
        xj = _gather(h, src3).reshape(E, W)
        msg = _msg(xj, w, sel).reshape(NW, NCH, CH, W)
        agg2 = _scatter(msg, dst3, zeros)
        if d < DEPTH - 1:
            h = _update(agg2, dinv, h, root, conv_br, relu=True)
        else:
            out = _final(agg2, dinv, h, root, conv_br, fc2_W,
                         fc2_b.reshape(1, 1))
    return out
